# baseline (speedup 1.0000x reference)
import jax
import jax.numpy as jnp
from jax.experimental import pallas as pl
from jax.experimental.pallas import tpu as pltpu

HEADS = 32
EPS = 1e-5

TM_A = 256
TM_B = 1024
TN_B = 1024


def _fc1_ln_kernel(x_ref, w1_ref, gamma_ref, beta_ref, w2h_ref,
                   h_ref, heads_ref):
    h = jnp.dot(x_ref[...].astype(jnp.bfloat16), w1_ref[...],
                preferred_element_type=jnp.float32)
    mu = jnp.mean(h, axis=-1, keepdims=True)
    var = jnp.mean((h - mu) ** 2, axis=-1, keepdims=True)
    hn = (h - mu) * jax.lax.rsqrt(var + EPS) * gamma_ref[...] + beta_ref[...]
    hnb = hn.astype(jnp.bfloat16)
    h_ref[...] = hnb
    heads_ref[...] = jnp.dot(hnb, w2h_ref[...],
                             preferred_element_type=jnp.float32)


def _cast_bf16_kernel(src_ref, dst_ref):
    dst_ref[...] = src_ref[...].astype(jnp.bfloat16)


def _fc2_kernel(h_ref, w2n_ref, out_ref):
    out_ref[...] = jax.lax.dot_general(
        h_ref[...], w2n_ref[...],
        (((1,), (1,)), ((), ())),
        preferred_element_type=jnp.float32)


def kernel(x, W1, gamma, beta, W2):
    n_tokens, embed = x.shape
    hidden = W1.shape[0]
    n_out = W2.shape[0]
    n_neurons = n_out - HEADS

    W1T = W1.T.astype(jnp.bfloat16)
    RB = 1024
    W2n = pl.pallas_call(
        _cast_bf16_kernel,
        grid=(n_neurons // RB,),
        in_specs=[pl.BlockSpec((RB, hidden), lambda r: (r, 0))],
        out_specs=pl.BlockSpec((RB, hidden), lambda r: (r, 0)),
        out_shape=jax.ShapeDtypeStruct((n_neurons, hidden), jnp.bfloat16),
        compiler_params=pltpu.CompilerParams(
            dimension_semantics=("parallel",)),
    )(W2[:n_neurons, :])
    W2hT = W2[n_neurons:, :].T.astype(jnp.bfloat16)
    gamma2 = gamma.reshape(1, hidden)
    beta2 = beta.reshape(1, hidden)

    grid_a = (n_tokens // TM_A,)
    h, heads = pl.pallas_call(
        _fc1_ln_kernel,
        grid=grid_a,
        in_specs=[
            pl.BlockSpec((TM_A, embed), lambda i: (i, 0)),
            pl.BlockSpec((embed, hidden), lambda i: (0, 0)),
            pl.BlockSpec((1, hidden), lambda i: (0, 0)),
            pl.BlockSpec((1, hidden), lambda i: (0, 0)),
            pl.BlockSpec((hidden, HEADS), lambda i: (0, 0)),
        ],
        out_specs=[
            pl.BlockSpec((TM_A, hidden), lambda i: (i, 0)),
            pl.BlockSpec((TM_A, HEADS), lambda i: (i, 0)),
        ],
        out_shape=[
            jax.ShapeDtypeStruct((n_tokens, hidden), jnp.bfloat16),
            jax.ShapeDtypeStruct((n_tokens, HEADS), jnp.float32),
        ],
    )(x, W1T, gamma2, beta2, W2hT)

    grid_b = (n_tokens // TM_B, n_neurons // TN_B)
    neurons = pl.pallas_call(
        _fc2_kernel,
        grid=grid_b,
        in_specs=[
            pl.BlockSpec((TM_B, hidden), lambda i, j: (i, 0)),
            pl.BlockSpec((TN_B, hidden), lambda i, j: (j, 0)),
        ],
        out_specs=pl.BlockSpec((TM_B, TN_B), lambda i, j: (i, j)),
        out_shape=jax.ShapeDtypeStruct((n_tokens, n_neurons), jnp.float32),
        compiler_params=pltpu.CompilerParams(
            dimension_semantics=("parallel", "parallel")),
    )(h, W2n)

    return (neurons, heads)


# speedup vs baseline: 1.0401x; 1.0401x over previous
import jax
import jax.numpy as jnp
from jax.experimental import pallas as pl
from jax.experimental.pallas import tpu as pltpu

HEADS = 32
EPS = 1e-5

TM_A = 256
TM_B = 2048
TN_B = 512


def _fc1_ln_kernel(x_ref, w1_ref, gamma_ref, beta_ref, w2h_ref,
                   h_ref, heads_ref):
    h = jnp.dot(x_ref[...].astype(jnp.bfloat16), w1_ref[...],
                preferred_element_type=jnp.float32)
    mu = jnp.mean(h, axis=-1, keepdims=True)
    var = jnp.mean((h - mu) ** 2, axis=-1, keepdims=True)
    hn = (h - mu) * jax.lax.rsqrt(var + EPS) * gamma_ref[...] + beta_ref[...]
    hnb = hn.astype(jnp.bfloat16)
    h_ref[...] = hnb
    heads_ref[...] = jnp.dot(hnb, w2h_ref[...],
                             preferred_element_type=jnp.float32)


def _fc2_kernel(h_ref, w2n_ref, out_ref):
    out_ref[...] = jax.lax.dot_general(
        h_ref[...], w2n_ref[...],
        (((1,), (1,)), ((), ())),
        preferred_element_type=jnp.float32)


def kernel(x, W1, gamma, beta, W2):
    n_tokens, embed = x.shape
    hidden = W1.shape[0]
    n_out = W2.shape[0]
    n_neurons = n_out - HEADS

    W1T = W1.T.astype(jnp.bfloat16)
    W2n = W2[:n_neurons, :].astype(jnp.bfloat16)
    W2hT = W2[n_neurons:, :].T.astype(jnp.bfloat16)
    gamma2 = gamma.reshape(1, hidden)
    beta2 = beta.reshape(1, hidden)

    grid_a = (n_tokens // TM_A,)
    h, heads = pl.pallas_call(
        _fc1_ln_kernel,
        grid=grid_a,
        in_specs=[
            pl.BlockSpec((TM_A, embed), lambda i: (i, 0)),
            pl.BlockSpec((embed, hidden), lambda i: (0, 0)),
            pl.BlockSpec((1, hidden), lambda i: (0, 0)),
            pl.BlockSpec((1, hidden), lambda i: (0, 0)),
            pl.BlockSpec((hidden, HEADS), lambda i: (0, 0)),
        ],
        out_specs=[
            pl.BlockSpec((TM_A, hidden), lambda i: (i, 0)),
            pl.BlockSpec((TM_A, HEADS), lambda i: (i, 0)),
        ],
        out_shape=[
            jax.ShapeDtypeStruct((n_tokens, hidden), jnp.bfloat16),
            jax.ShapeDtypeStruct((n_tokens, HEADS), jnp.float32),
        ],
    )(x, W1T, gamma2, beta2, W2hT)

    grid_b = (n_tokens // TM_B, n_neurons // TN_B)
    neurons = pl.pallas_call(
        _fc2_kernel,
        grid=grid_b,
        in_specs=[
            pl.BlockSpec((TM_B, hidden), lambda i, j: (i, 0)),
            pl.BlockSpec((TN_B, hidden), lambda i, j: (j, 0)),
        ],
        out_specs=pl.BlockSpec((TM_B, TN_B), lambda i, j: (i, j)),
        out_shape=jax.ShapeDtypeStruct((n_tokens, n_neurons), jnp.float32),
        compiler_params=pltpu.CompilerParams(
            dimension_semantics=("parallel", "parallel")),
    )(h, W2n)

    return (neurons, heads)
